# baseline (speedup 1.0000x reference)
import jax
import jax.numpy as jnp
from jax.experimental import pallas as pl
from jax.experimental.pallas import tpu as pltpu

_MARGIN = 1.35
_CHUNKS = (2000, 23600, 23600, 23600, 23600, 3600)
_MC = max(_CHUNKS)
_OFFS = tuple(sum(_CHUNKS[:i]) for i in range(len(_CHUNKS)))


def _body(lab_ref, x_hbm, o_hbm, ibuf, obuf, isems, osems):
    n = len(_CHUNKS)
    lab = lab_ref[...]

    def in_copy(k):
        return pltpu.make_async_copy(
            x_hbm.at[pl.ds(_OFFS[k], _CHUNKS[k])],
            ibuf.at[k % 2, pl.ds(0, _CHUNKS[k])],
            isems.at[k % 2],
        )

    def out_copy(k):
        return pltpu.make_async_copy(
            obuf.at[k % 2, pl.ds(0, _CHUNKS[k])],
            o_hbm.at[pl.ds(_OFFS[k], _CHUNKS[k])],
            osems.at[k % 2],
        )

    in_copy(0).start()
    in_copy(1).start()
    for k in range(n):
        in_copy(k).wait()
        if k >= 2:
            out_copy(k - 2).wait()
        x = ibuf[k % 2, pl.ds(0, _CHUNKS[k]), :]
        rows = jax.lax.broadcasted_iota(jnp.int32, x.shape, 0) + _OFFS[k]
        obuf[k % 2, pl.ds(0, _CHUNKS[k]), :] = jnp.where(
            rows == lab, x * _MARGIN, x
        )
        out_copy(k).start()
        if k + 2 < n:
            in_copy(k + 2).start()
    out_copy(n - 2).wait()
    out_copy(n - 1).wait()


def kernel(logits, label):
    b, v = logits.shape
    xt = logits.T
    lab = label.astype(jnp.int32).reshape(1, b)
    out_t = pl.pallas_call(
        _body,
        in_specs=[
            pl.BlockSpec(memory_space=pltpu.MemorySpace.VMEM),
            pl.BlockSpec(memory_space=pltpu.MemorySpace.HBM),
        ],
        out_specs=pl.BlockSpec(memory_space=pltpu.MemorySpace.HBM),
        out_shape=jax.ShapeDtypeStruct((v, b), logits.dtype),
        scratch_shapes=[
            pltpu.VMEM((2, _MC, b), logits.dtype),
            pltpu.VMEM((2, _MC, b), logits.dtype),
            pltpu.SemaphoreType.DMA((2,)),
            pltpu.SemaphoreType.DMA((2,)),
        ],
    )(lab, xt)
    return out_t.T


# speedup vs baseline: 1.0413x; 1.0413x over previous
import jax
import jax.numpy as jnp
from jax.experimental import pallas as pl

_MARGIN = 1.35
_VB = 29992


def _scale_body(lab_ref, x_ref, o_ref):
    i = pl.program_id(0)
    x = x_ref[...]
    rows = jax.lax.broadcasted_iota(jnp.int32, x.shape, 0) + i * _VB
    o_ref[...] = jnp.where(rows == lab_ref[...], x * _MARGIN, x)


def kernel(logits, label):
    b, v = logits.shape
    xt = logits.T
    lab = label.astype(jnp.int32).reshape(1, b)
    out_t = pl.pallas_call(
        _scale_body,
        grid=(pl.cdiv(v, _VB),),
        in_specs=[
            pl.BlockSpec((1, b), lambda i: (0, 0)),
            pl.BlockSpec((_VB, b), lambda i: (i, 0)),
        ],
        out_specs=pl.BlockSpec((_VB, b), lambda i: (i, 0)),
        out_shape=jax.ShapeDtypeStruct((v, b), logits.dtype),
    )(lab, xt)
    return out_t.T
